# baseline (speedup 1.0000x reference)
import functools

import jax
import jax.numpy as jnp
from jax import lax
from jax.experimental import pallas as pl
from jax.experimental.pallas import tpu as pltpu
from jax.experimental.pallas import tpu_sc as plsc

N0, D0 = 100000, 256
N1, D1 = 50000, 128
NM0 = N0 // 10
NM1 = N1 // 10

NS, L = 16, 16
NW = NS

CH = 64
K0 = 10
K1 = 5

_DMA_CHUNKS = 4


def _dma_copy_body(src_hbm, dst_hbm, sem):
    n = src_hbm.shape[0]
    c = n // _DMA_CHUNKS
    for k in range(_DMA_CHUNKS):
        pltpu.make_async_copy(
            src_hbm.at[pl.ds(k * c, c)], dst_hbm.at[pl.ds(k * c, c)], sem
        ).start()
    for k in range(_DMA_CHUNKS):
        pltpu.make_async_copy(
            src_hbm.at[pl.ds(k * c, c)], dst_hbm.at[pl.ds(k * c, c)], sem
        ).wait()


def _copy(feat):
    n, d = feat.shape
    return pl.pallas_call(
        _dma_copy_body,
        in_specs=[pl.BlockSpec(memory_space=pl.ANY)],
        out_specs=pl.BlockSpec(memory_space=pl.ANY),
        out_shape=jax.ShapeDtypeStruct((n, d), jnp.float32),
        scratch_shapes=[pltpu.SemaphoreType.DMA],
    )(feat)


@functools.lru_cache(maxsize=None)
def _fixup_kernel():
    mesh = plsc.VectorSubcoreMesh(
        core_axis_name="c", subcore_axis_name="s", num_cores=1, num_subcores=NS
    )
    return pl.kernel(
        _fixup_body,
        out_type=(),
        mesh=mesh,
        compiler_params=pltpu.CompilerParams(needs_layout_passes=False),
        scratch_types=[
            pltpu.VMEM((K0, CH), jnp.int32),
            pltpu.VMEM((K1, CH), jnp.int32),
            pltpu.VMEM((CH, D0), jnp.float32),
            pltpu.VMEM((CH, D1), jnp.float32),
            pltpu.SemaphoreType.DMA,
        ],
    )


def _fixup_body(idx0_hbm, idx1_hbm, tok0_hbm, tok1_hbm, out0_hbm, out1_hbm,
                idx0_v, idx1_v, rows0_v, rows1_v, sem):
    w = lax.axis_index("s")
    pltpu.sync_copy(idx0_hbm.at[w], idx0_v)
    pltpu.sync_copy(idx1_hbm.at[w], idx1_v)
    pltpu.sync_copy(tok0_hbm, rows0_v.at[pl.ds(0, 1)])
    pltpu.sync_copy(tok1_hbm, rows1_v.at[pl.ds(0, 1)])

    row0 = tuple(rows0_v[0, pl.ds(c * L, L)] for c in range(D0 // L))
    row1 = tuple(rows1_v[0, pl.ds(c * L, L)] for c in range(D1 // L))

    def rep0_body(r, carry):
        for c in range(D0 // L):
            rows0_v[r, pl.ds(c * L, L)] = row0[c]
        return carry

    lax.fori_loop(1, CH, rep0_body, 0)

    def rep1_body(r, carry):
        for c in range(D1 // L):
            rows1_v[r, pl.ds(c * L, L)] = row1[c]
        return carry

    lax.fori_loop(1, CH, rep1_body, 0)

    copies = [
        pltpu.async_copy(rows0_v, out0_hbm.at[idx0_v.at[j]], sem)
        for j in range(K0)
    ] + [
        pltpu.async_copy(rows1_v, out1_hbm.at[idx1_v.at[j]], sem)
        for j in range(K1)
    ]
    for cp in copies:
        cp.wait()


def _pad_idx_3d(idx, k):
    pad = NW * k * CH - idx.shape[0]
    return jnp.concatenate(
        [idx, jnp.broadcast_to(idx[:1], (pad,))]
    ).reshape(NW, k, CH)


def kernel(feat0, feat1, token0, token1, mask_nodes0, keep_nodes0,
           mask_nodes1, keep_nodes1):
    idx0_3d = _pad_idx_3d(mask_nodes0, K0)
    idx1_3d = _pad_idx_3d(mask_nodes1, K1)

    out0 = _copy(feat0)
    out1 = _copy(feat1)
    ref0 = jax.new_ref(out0)
    ref1 = jax.new_ref(out1)
    _fixup_kernel()(idx0_3d, idx1_3d, token0, token1, ref0, ref1)
    masked0 = ref0[...]
    masked1 = ref1[...]
    return (masked0, masked1, mask_nodes0, keep_nodes0, mask_nodes1, keep_nodes1)


# speedup vs baseline: 1.0036x; 1.0036x over previous
import functools

import jax
import jax.numpy as jnp
from jax import lax
from jax.experimental import pallas as pl
from jax.experimental.pallas import tpu as pltpu
from jax.experimental.pallas import tpu_sc as plsc

N0, D0 = 100000, 256
N1, D1 = 50000, 128
NM0 = N0 // 10
NM1 = N1 // 10

NS, L = 16, 16
NW = NS

CH = 64
K0 = 10
K1 = 5

_DMA_CHUNKS = 4


def _dma_copy_body(src_hbm, dst_hbm, sem):
    n = src_hbm.shape[0]
    c = n // _DMA_CHUNKS
    for k in range(_DMA_CHUNKS):
        pltpu.make_async_copy(
            src_hbm.at[pl.ds(k * c, c)], dst_hbm.at[pl.ds(k * c, c)], sem
        ).start()
    for k in range(_DMA_CHUNKS):
        pltpu.make_async_copy(
            src_hbm.at[pl.ds(k * c, c)], dst_hbm.at[pl.ds(k * c, c)], sem
        ).wait()


def _copy(feat):
    n, d = feat.shape
    return pl.pallas_call(
        _dma_copy_body,
        in_specs=[pl.BlockSpec(memory_space=pl.ANY)],
        out_specs=pl.BlockSpec(memory_space=pl.ANY),
        out_shape=jax.ShapeDtypeStruct((n, d), jnp.float32),
        scratch_shapes=[pltpu.SemaphoreType.DMA],
    )(feat)


@functools.lru_cache(maxsize=None)
def _fixup_kernel(d, k):
    mesh = plsc.VectorSubcoreMesh(
        core_axis_name="c", subcore_axis_name="s", num_cores=1, num_subcores=NS
    )
    return pl.kernel(
        _make_fixup_body(d, k),
        out_type=(),
        mesh=mesh,
        compiler_params=pltpu.CompilerParams(needs_layout_passes=False),
        scratch_types=[
            pltpu.VMEM((k, CH), jnp.int32),
            pltpu.VMEM((CH, d), jnp.float32),
            pltpu.SemaphoreType.DMA,
        ],
    )


def _make_fixup_body(d, k):
    def body(idx_hbm, tok_hbm, out_hbm, idx_v, rows_v, sem):
        w = lax.axis_index("s")
        pltpu.sync_copy(idx_hbm.at[w], idx_v)
        pltpu.sync_copy(tok_hbm, rows_v.at[pl.ds(0, 1)])
        row = tuple(rows_v[0, pl.ds(c * L, L)] for c in range(d // L))

        def rep_body(r, carry):
            for c in range(d // L):
                rows_v[r, pl.ds(c * L, L)] = row[c]
            return carry

        lax.fori_loop(1, CH, rep_body, 0)

        copies = [
            pltpu.async_copy(rows_v, out_hbm.at[idx_v.at[j]], sem)
            for j in range(k)
        ]
        for cp in copies:
            cp.wait()

    return body


def _pad_idx_3d(idx, k):
    pad = NW * k * CH - idx.shape[0]
    return jnp.concatenate(
        [idx, jnp.broadcast_to(idx[:1], (pad,))]
    ).reshape(NW, k, CH)


def kernel(feat0, feat1, token0, token1, mask_nodes0, keep_nodes0,
           mask_nodes1, keep_nodes1):
    idx0_3d = _pad_idx_3d(mask_nodes0, K0)
    idx1_3d = _pad_idx_3d(mask_nodes1, K1)

    out0 = _copy(feat0)
    out1 = _copy(feat1)
    ref0 = jax.new_ref(out0)
    ref1 = jax.new_ref(out1)
    _fixup_kernel(D0, K0)(idx0_3d, token0, ref0)
    _fixup_kernel(D1, K1)(idx1_3d, token1, ref1)
    masked0 = ref0[...]
    masked1 = ref1[...]
    return (masked0, masked1, mask_nodes0, keep_nodes0, mask_nodes1, keep_nodes1)


# speedup vs baseline: 31.5065x; 31.3949x over previous
import functools

import jax
import jax.numpy as jnp
from jax import lax
from jax.experimental import pallas as pl
from jax.experimental.pallas import tpu as pltpu
from jax.experimental.pallas import tpu_sc as plsc

N0, D0 = 100000, 256
N1, D1 = 50000, 128
NM0 = N0 // 10
NM1 = N1 // 10

NS, L = 16, 16
NW = NS

CH = 64
K0 = 10
K1 = 5


def _grid_copy_body(src_ref, dst_ref):
    dst_ref[...] = src_ref[...]


def _copy(feat, block):
    n, d = feat.shape
    return pl.pallas_call(
        _grid_copy_body,
        grid=(n // block,),
        in_specs=[pl.BlockSpec((block, d), lambda i: (i, 0))],
        out_specs=pl.BlockSpec((block, d), lambda i: (i, 0)),
        out_shape=jax.ShapeDtypeStruct((n, d), jnp.float32),
    )(feat)


def _sc_mesh():
    return plsc.VectorSubcoreMesh(
        core_axis_name="c", subcore_axis_name="s", num_cores=1, num_subcores=NS
    )


@functools.lru_cache(maxsize=None)
def _warm_kernel():
    return pl.kernel(
        _warm_body,
        out_type=(),
        mesh=_sc_mesh(),
        compiler_params=pltpu.CompilerParams(
            needs_layout_passes=False, has_side_effects=True
        ),
        scratch_types=[pltpu.VMEM((L,), jnp.float32)],
    )


def _warm_body(buf):
    buf[...] = jnp.zeros((L,), jnp.float32)


@functools.lru_cache(maxsize=None)
def _fixup_kernel():
    return pl.kernel(
        _fixup_body,
        out_type=(),
        mesh=_sc_mesh(),
        compiler_params=pltpu.CompilerParams(needs_layout_passes=False),
        scratch_types=[
            pltpu.VMEM((K0, CH), jnp.int32),
            pltpu.VMEM((K1, CH), jnp.int32),
            pltpu.VMEM((CH, D0), jnp.float32),
            pltpu.VMEM((CH, D1), jnp.float32),
            pltpu.SemaphoreType.DMA,
        ],
    )


def _fixup_body(idx0_hbm, idx1_hbm, tok0_hbm, tok1_hbm, out0_hbm, out1_hbm,
                idx0_v, idx1_v, rows0_v, rows1_v, sem):
    w = lax.axis_index("s")
    pltpu.sync_copy(idx0_hbm.at[w], idx0_v)
    pltpu.sync_copy(idx1_hbm.at[w], idx1_v)
    pltpu.sync_copy(tok0_hbm, rows0_v.at[pl.ds(0, 1)])
    pltpu.sync_copy(tok1_hbm, rows1_v.at[pl.ds(0, 1)])

    row0 = tuple(rows0_v[0, pl.ds(c * L, L)] for c in range(D0 // L))
    row1 = tuple(rows1_v[0, pl.ds(c * L, L)] for c in range(D1 // L))

    def rep0_body(r, carry):
        for c in range(D0 // L):
            rows0_v[r, pl.ds(c * L, L)] = row0[c]
        return carry

    lax.fori_loop(1, CH, rep0_body, 0)

    def rep1_body(r, carry):
        for c in range(D1 // L):
            rows1_v[r, pl.ds(c * L, L)] = row1[c]
        return carry

    lax.fori_loop(1, CH, rep1_body, 0)

    copies = [
        pltpu.async_copy(rows0_v, out0_hbm.at[idx0_v.at[j]], sem)
        for j in range(K0)
    ] + [
        pltpu.async_copy(rows1_v, out1_hbm.at[idx1_v.at[j]], sem)
        for j in range(K1)
    ]
    for cp in copies:
        cp.wait()


def _pad_idx_3d(idx, k):
    pad = NW * k * CH - idx.shape[0]
    return jnp.concatenate(
        [idx, jnp.broadcast_to(idx[:1], (pad,))]
    ).reshape(NW, k, CH)


def kernel(feat0, feat1, token0, token1, mask_nodes0, keep_nodes0,
           mask_nodes1, keep_nodes1):
    idx0_3d = _pad_idx_3d(mask_nodes0, K0)
    idx1_3d = _pad_idx_3d(mask_nodes1, K1)

    _warm_kernel()()
    out0 = _copy(feat0, 10000)
    out1 = _copy(feat1, 10000)
    ref0 = jax.new_ref(out0)
    ref1 = jax.new_ref(out1)
    _fixup_kernel()(idx0_3d, idx1_3d, token0, token1, ref0, ref1)
    masked0 = ref0[...]
    masked1 = ref1[...]
    return (masked0, masked1, mask_nodes0, keep_nodes0, mask_nodes1, keep_nodes1)


# speedup vs baseline: 31.5611x; 1.0017x over previous
import functools

import jax
import jax.numpy as jnp
from jax import lax
from jax.experimental import pallas as pl
from jax.experimental.pallas import tpu as pltpu
from jax.experimental.pallas import tpu_sc as plsc

N0, D0 = 100000, 256
N1, D1 = 50000, 128
NM0 = N0 // 10
NM1 = N1 // 10

NS, L = 16, 16
NW = NS

CH = 64
K0 = 10
K1 = 5


def _grid_copy_body(src_ref, dst_ref):
    dst_ref[...] = src_ref[...]


def _copy(feat, block):
    n, d = feat.shape
    return pl.pallas_call(
        _grid_copy_body,
        grid=(n // block,),
        in_specs=[pl.BlockSpec((block, d), lambda i: (i, 0))],
        out_specs=pl.BlockSpec((block, d), lambda i: (i, 0)),
        out_shape=jax.ShapeDtypeStruct((n, d), jnp.float32),
    )(feat)


def _sc_mesh():
    return plsc.VectorSubcoreMesh(
        core_axis_name="c", subcore_axis_name="s", num_cores=1, num_subcores=NS
    )


@functools.lru_cache(maxsize=None)
def _warm_kernel():
    return pl.kernel(
        _warm_body,
        out_type=(),
        mesh=_sc_mesh(),
        compiler_params=pltpu.CompilerParams(
            needs_layout_passes=False,
            has_side_effects=True,
            skip_device_barrier=True,
        ),
        scratch_types=[pltpu.VMEM((L,), jnp.float32)],
    )


def _warm_body(buf):
    buf[...] = jnp.zeros((L,), jnp.float32)


@functools.lru_cache(maxsize=None)
def _fixup_kernel():
    return pl.kernel(
        _fixup_body,
        out_type=(),
        mesh=_sc_mesh(),
        compiler_params=pltpu.CompilerParams(
            needs_layout_passes=False,
            skip_device_barrier=True,
            disable_bounds_checks=True,
            disable_semaphore_checks=True,
        ),
        scratch_types=[
            pltpu.VMEM((K0, CH), jnp.int32),
            pltpu.VMEM((K1, CH), jnp.int32),
            pltpu.VMEM((CH, D0), jnp.float32),
            pltpu.VMEM((CH, D1), jnp.float32),
            pltpu.SemaphoreType.DMA,
        ],
    )


def _fixup_body(idx0_hbm, idx1_hbm, tok0_hbm, tok1_hbm, out0_hbm, out1_hbm,
                idx0_v, idx1_v, rows0_v, rows1_v, sem):
    w = lax.axis_index("s")
    pltpu.sync_copy(idx0_hbm.at[w], idx0_v)
    pltpu.sync_copy(idx1_hbm.at[w], idx1_v)
    pltpu.sync_copy(tok0_hbm, rows0_v.at[pl.ds(0, 1)])
    pltpu.sync_copy(tok1_hbm, rows1_v.at[pl.ds(0, 1)])

    row0 = tuple(rows0_v[0, pl.ds(c * L, L)] for c in range(D0 // L))
    row1 = tuple(rows1_v[0, pl.ds(c * L, L)] for c in range(D1 // L))

    def rep0_body(r, carry):
        for c in range(D0 // L):
            rows0_v[r, pl.ds(c * L, L)] = row0[c]
        return carry

    lax.fori_loop(1, CH, rep0_body, 0)

    def rep1_body(r, carry):
        for c in range(D1 // L):
            rows1_v[r, pl.ds(c * L, L)] = row1[c]
        return carry

    lax.fori_loop(1, CH, rep1_body, 0)

    copies = [
        pltpu.async_copy(rows0_v, out0_hbm.at[idx0_v.at[j]], sem)
        for j in range(K0)
    ] + [
        pltpu.async_copy(rows1_v, out1_hbm.at[idx1_v.at[j]], sem)
        for j in range(K1)
    ]
    for cp in copies:
        cp.wait()


def _pad_idx_3d(idx, k):
    pad = NW * k * CH - idx.shape[0]
    return jnp.concatenate(
        [idx, jnp.broadcast_to(idx[:1], (pad,))]
    ).reshape(NW, k, CH)


def kernel(feat0, feat1, token0, token1, mask_nodes0, keep_nodes0,
           mask_nodes1, keep_nodes1):
    idx0_3d = _pad_idx_3d(mask_nodes0, K0)
    idx1_3d = _pad_idx_3d(mask_nodes1, K1)

    _warm_kernel()()
    out0 = _copy(feat0, 10000)
    out1 = _copy(feat1, 10000)
    ref0 = jax.new_ref(out0)
    ref1 = jax.new_ref(out1)
    _fixup_kernel()(idx0_3d, idx1_3d, token0, token1, ref0, ref1)
    masked0 = ref0[...]
    masked1 = ref1[...]
    return (masked0, masked1, mask_nodes0, keep_nodes0, mask_nodes1, keep_nodes1)
